# baseline (speedup 1.0000x reference)
import functools

import jax
import jax.numpy as jnp
from jax import lax
from jax.experimental import pallas as pl
from jax.experimental.pallas import tpu as pltpu
from jax.experimental.pallas import tpu_sc as plsc

LANES = 16
SLAB = 128
SLABS_PER_CHUNK = 8
CHUNK = SLAB * SLABS_PER_CHUNK


def _build_lookup(s_dim: int, b_dim: int, embed: int, num_workers: int):
    n_slabs = s_dim * (b_dim // SLAB)
    slabs_per_worker = n_slabs // num_workers
    n_chunks = slabs_per_worker // SLABS_PER_CHUNK
    tb_dim = b_dim // SLAB
    te_dim = embed // 8

    mesh = plsc.VectorSubcoreMesh(core_axis_name="c", subcore_axis_name="s")

    @functools.partial(
        pl.kernel,
        mesh=mesh,
        compiler_params=pltpu.CompilerParams(
            use_tc_tiling_on_sc=False, needs_layout_passes=False),
        out_type=jax.ShapeDtypeStruct((s_dim, te_dim, tb_dim, 8, SLAB),
                                      jnp.float32),
        scratch_types=[
            pltpu.VMEM((SLABS_PER_CHUNK, SLAB), jnp.int32),
            pltpu.VMEM((CHUNK, embed), jnp.float32),
            pltpu.VMEM((SLABS_PER_CHUNK, te_dim, 8, SLAB), jnp.float32),
            pltpu.SemaphoreType.DMA,
            pltpu.SemaphoreType.DMA,
        ],
    )
    def lookup(table_hbm, idx_hbm, out_hbm, idx_v, raw_v, tout_v, gsem, wsem):
        n_cores = lax.axis_size("c")
        wid = lax.axis_index("s") * n_cores + lax.axis_index("c")
        slab_base = wid * slabs_per_worker
        iot = lax.iota(jnp.int32, LANES)
        zeros16 = jnp.zeros((LANES,), jnp.float32)

        def do_chunk(g, carry):
            chunk_slab = slab_base + g * SLABS_PER_CHUNK
            pltpu.sync_copy(idx_hbm.at[pl.ds(chunk_slab, SLABS_PER_CHUNK)],
                            idx_v)
            gathers = [
                pltpu.async_copy(
                    table_hbm.at[idx_v.at[j]],
                    raw_v.at[pl.ds(j * SLAB, SLAB)],
                    gsem,
                )
                for j in range(SLABS_PER_CHUNK)
            ]
            for cp in gathers:
                cp.wait()

            def transpose_slab(j, c):
                def transpose_group(bg, c2):
                    v = idx_v[j, pl.ds(bg * LANES, LANES)]
                    m = v != 0
                    rows = iot + (j * SLAB + bg * LANES)
                    for e in range(embed):
                        col = jnp.full((LANES,), e, jnp.int32)
                        vals = plsc.load_gather(raw_v, [rows, col])
                        tout_v[j, e // 8, e % 8,
                               pl.ds(bg * LANES, LANES)] = jnp.where(
                                   m, vals, zeros16)
                    return c2

                return lax.fori_loop(0, SLAB // LANES, transpose_group, c)

            lax.fori_loop(0, SLABS_PER_CHUNK, transpose_slab, 0)

            writes = []
            for j in range(SLABS_PER_CHUNK):
                sj = (chunk_slab + j) // tb_dim
                tbj = (chunk_slab + j) % tb_dim
                for te in range(te_dim):
                    writes.append(pltpu.async_copy(
                        tout_v.at[j, te], out_hbm.at[sj, te, tbj], wsem))
            for cp in writes:
                cp.wait()
            return carry

        lax.fori_loop(0, n_chunks, do_chunk, 0)

    return lookup


def kernel(instruction, table):
    b, s = instruction.shape
    vocab, embed = table.shape
    idx = instruction.astype(jnp.int32).T.reshape(s * (b // SLAB), SLAB)
    info = plsc.get_sparse_core_info()
    num_workers = info.num_cores * info.num_subcores
    out5 = _build_lookup(s, b, embed, num_workers)(table, idx)
    return out5.transpose(2, 4, 0, 1, 3).reshape(b, s, embed)
